# baseline (speedup 1.0000x reference)
import jax
import jax.numpy as jnp
from jax.experimental import pallas as pl
from jax.experimental.pallas import tpu as pltpu

N_NODES = 50000
N_EDGES = 800000
D_NODE = 16
HID = 128
HEADS = 4
HDIM = 32

NODE_BLK = 2000


def _enc_body(nf_ref, types_ref, w_ref, b_ref, temb_ref, out_ref):
    x = nf_ref[...] @ w_ref[...] + b_ref[...]
    t = types_ref[...]
    for k in range(3):
        x = x + jnp.where((t == k), temb_ref[k, :][None, :], 0.0)
    out_ref[...] = x


def _encode_nodes(nf, types, W, b, temb):
    n = nf.shape[0]
    grid = n // NODE_BLK
    return pl.pallas_call(
        _enc_body,
        grid=(grid,),
        in_specs=[
            pl.BlockSpec((NODE_BLK, D_NODE), lambda i: (i, 0)),
            pl.BlockSpec((NODE_BLK, 1), lambda i: (i, 0)),
            pl.BlockSpec((D_NODE, HID), lambda i: (0, 0)),
            pl.BlockSpec((1, HID), lambda i: (0, 0)),
            pl.BlockSpec((3, HID), lambda i: (0, 0)),
        ],
        out_specs=pl.BlockSpec((NODE_BLK, HID), lambda i: (i, 0)),
        out_shape=jax.ShapeDtypeStruct((n, HID), jnp.float32),
    )(nf, types.reshape(n, 1).astype(jnp.int32), W, b.reshape(1, HID), temb)


def _layernorm(x, g, b):
    mu = jnp.mean(x, -1, keepdims=True)
    var = jnp.var(x, -1, keepdims=True)
    return (x - mu) / jnp.sqrt(var + 1e-5) * g + b


def _gat_layer(x, src, dst, W, a_src, a_dst):
    N = x.shape[0]
    h = (x @ W).reshape(N, HEADS, HDIM)
    e = jax.nn.leaky_relu(
        jnp.sum(h * a_src[None], -1)[src] + jnp.sum(h * a_dst[None], -1)[dst], 0.2)
    m = jax.ops.segment_max(e, dst, num_segments=N)
    m = jnp.where(jnp.isfinite(m), m, 0.0)
    ex = jnp.exp(e - m[dst])
    den = jax.ops.segment_sum(ex, dst, num_segments=N) + 1e-9
    msg = h[src] * (ex / den[dst])[:, :, None]
    out = jax.ops.segment_sum(msg, dst, num_segments=N).reshape(N, HID)
    return jax.nn.elu(out) + x


def kernel(node_features, edge_features, node_positions, node_radii,
           edge_index, node_types, params):
    p = params
    src, dst = edge_index[0], edge_index[1]
    N = node_features.shape[0]
    x = _encode_nodes(node_features, node_types, p['enc_node_W'],
                      p['enc_node_b'], p['type_emb'])
    eh = jax.nn.relu(edge_features @ p['enc_edge_W'] + p['enc_edge_b'])
    topo = x
    for i in range(3):
        topo = _gat_layer(topo, src, dst, p['gat%d_W' % i],
                          p['gat%d_asrc' % i], p['gat%d_adst' % i])
    geo = jnp.concatenate([node_positions[src] - node_positions[dst],
                           (node_radii[src] - node_radii[dst])[:, None]], -1)
    gate = jax.nn.sigmoid(geo @ p['geo_W'] + p['geo_b'])
    agg = jax.ops.segment_sum(topo[src] * gate, dst, num_segments=N)
    anat = jax.nn.relu(agg @ p['anat_W'] + p['anat_b']) + topo
    fused = _layernorm(jax.nn.relu(
        jnp.concatenate([topo, anat], -1) @ p['fuse_W'] + p['fuse_b']),
        p['ln_g'], p['ln_b'])
    node_op = jax.nn.relu(fused @ p['nop_W1'] + p['nop_b1']) @ p['nop_W2'] + p['nop_b2']
    node_corr = jax.nn.relu(fused @ p['ncr_W1'] + p['ncr_b1']) @ p['ncr_W2'] + p['ncr_b2']
    quality = jax.nn.sigmoid(jax.nn.relu(fused @ p['q_W1'] + p['q_b1']) @ p['q_W2'] + p['q_b2'])
    node_out = jnp.concatenate([node_op, node_corr, quality], -1)
    erep = eh + fused[src] + fused[dst]
    edge_op = jax.nn.relu(erep @ p['eop_W1'] + p['eop_b1']) @ p['eop_W2'] + p['eop_b2']
    return node_out, edge_op


# speedup vs baseline: 14.2599x; 14.2599x over previous
import functools

import jax
import jax.numpy as jnp
from jax import lax
from jax.experimental import pallas as pl
from jax.experimental.pallas import tpu as pltpu
from jax.experimental.pallas import tpu_sc as plsc

F32 = jnp.float32
I32 = jnp.int32

N_NODES = 50000
NP = 51200
N_EDGES = 800000
EP = 819200
EROWS = EP // 128
D_NODE = 16
D_EDGE = 8
HID = 128
HEADS = 4
HDIM = 32
NC16 = 8
CW = 16

BLK = 1600
EBLK = 2048

_MESH = plsc.VectorSubcoreMesh(core_axis_name="c", subcore_axis_name="s")
NEG_BIG = -3.0e38

_GDN = lax.GatherDimensionNumbers(
    offset_dims=(), collapsed_slice_dims=(0,), start_index_map=(0,))


def _lane_gather(x, perm):
    return lax.gather(x, perm[:, None], _GDN, (1,),
                      mode=lax.GatherScatterMode.PROMISE_IN_BOUNDS)


def _enc_body(nf_ref, types_ref, w_ref, b_ref, temb_ref, out_ref):
    x = jnp.dot(nf_ref[...], w_ref[...], preferred_element_type=F32, precision=lax.Precision.HIGHEST)
    x = x + b_ref[...]
    t = types_ref[...]
    for k in range(3):
        x = x + jnp.where(t == k, temb_ref[k, :][None, :], 0.0)
    out_ref[...] = x


def _tc_encode(nf, types, W, b, temb):
    return pl.pallas_call(
        _enc_body,
        grid=(NP // BLK,),
        in_specs=[
            pl.BlockSpec((BLK, D_NODE), lambda i: (i, 0)),
            pl.BlockSpec((BLK, 1), lambda i: (i, 0)),
            pl.BlockSpec((D_NODE, HID), lambda i: (0, 0)),
            pl.BlockSpec((1, HID), lambda i: (0, 0)),
            pl.BlockSpec((3, HID), lambda i: (0, 0)),
        ],
        out_specs=pl.BlockSpec((BLK, HID), lambda i: (i, 0)),
        out_shape=jax.ShapeDtypeStruct((NP, HID), F32),
    )(nf, types, W, b, temb)


def _proj_body(x_ref, w_ref, a8_ref, h_ref, s_ref):
    h = jnp.dot(x_ref[...], w_ref[...], preferred_element_type=F32, precision=lax.Precision.HIGHEST)
    for c in range(NC16):
        h_ref[c] = h[:, c * CW:(c + 1) * CW]
    s_ref[...] = jnp.dot(h, a8_ref[...], preferred_element_type=F32, precision=lax.Precision.HIGHEST)


def _tc_layer_proj(x, W, A8):
    return pl.pallas_call(
        _proj_body,
        grid=(NP // BLK,),
        in_specs=[
            pl.BlockSpec((BLK, HID), lambda i: (i, 0)),
            pl.BlockSpec((HID, HID), lambda i: (0, 0)),
            pl.BlockSpec((HID, 8), lambda i: (0, 0)),
        ],
        out_specs=[
            pl.BlockSpec((NC16, BLK, CW), lambda i: (0, i, 0)),
            pl.BlockSpec((BLK, 8), lambda i: (i, 0)),
        ],
        out_shape=[
            jax.ShapeDtypeStruct((NC16, NP, CW), F32),
            jax.ShapeDtypeStruct((NP, 8), F32),
        ],
    )(x, W, A8)


def _elu_res_body(o_ref, t_ref, out_ref, chunks_ref):
    o = jnp.concatenate([o_ref[c] for c in range(NC16)], axis=-1)
    r = jnp.where(o > 0, o, jnp.exp(jnp.minimum(o, 0.0)) - 1.0) + t_ref[...]
    out_ref[...] = r
    if chunks_ref is not None:
        for c in range(NC16):
            chunks_ref[c] = r[:, c * CW:(c + 1) * CW]


def _tc_elu_res(out_tbl, topo, with_chunks):
    body = _elu_res_body if with_chunks else (
        lambda o_ref, t_ref, out_ref: _elu_res_body(o_ref, t_ref, out_ref, None))
    out_specs = [pl.BlockSpec((BLK, HID), lambda i: (i, 0))]
    out_shape = [jax.ShapeDtypeStruct((NP, HID), F32)]
    if with_chunks:
        out_specs.append(pl.BlockSpec((NC16, BLK, CW), lambda i: (0, i, 0)))
        out_shape.append(jax.ShapeDtypeStruct((NC16, NP, CW), F32))
    return pl.pallas_call(
        body,
        grid=(NP // BLK,),
        in_specs=[
            pl.BlockSpec((NC16, BLK, CW), lambda i: (0, i, 0)),
            pl.BlockSpec((BLK, HID), lambda i: (i, 0)),
        ],
        out_specs=out_specs,
        out_shape=out_shape,
    )(out_tbl, topo)


def _geo_body(pos_ref, rad_ref, gw_ref, gb_ref, ua_ref, ub_ref):
    u = jnp.sum(pos_ref[...] * gw_ref[0:1, 0:3], axis=1, keepdims=True)
    u = u + rad_ref[...] * gw_ref[0:1, 3:4]
    ua_ref[...] = u + gb_ref[...]
    ub_ref[...] = -u


def _tc_geo(pos, rad, gw_row, gb):
    return pl.pallas_call(
        _geo_body,
        grid=(NP // BLK,),
        in_specs=[
            pl.BlockSpec((BLK, 3), lambda i: (i, 0)),
            pl.BlockSpec((BLK, 1), lambda i: (i, 0)),
            pl.BlockSpec((1, 4), lambda i: (0, 0)),
            pl.BlockSpec((1, 1), lambda i: (0, 0)),
        ],
        out_specs=[
            pl.BlockSpec((BLK, 1), lambda i: (i, 0)),
            pl.BlockSpec((BLK, 1), lambda i: (i, 0)),
        ],
        out_shape=[
            jax.ShapeDtypeStruct((NP, 1), F32),
            jax.ShapeDtypeStruct((NP, 1), F32),
        ],
    )(pos, rad, gw_row, gb)


def _fusion_body(agg_ref, topo_ref, aw_ref, ab_ref, fw1_ref, fw2_ref, fb_ref,
                 lg_ref, lb_ref, nw1_ref, nb1_ref, nw2_ref, nb2_ref,
                 cw1_ref, cb1_ref, cw2_ref, cb2_ref, qw1_ref, qb1_ref,
                 qw2_ref, qb2_ref, nout_ref, fchunks_ref):
    agg = jnp.concatenate([agg_ref[c] for c in range(NC16)], axis=-1)
    topo = topo_ref[...]
    anat = jax.nn.relu(jnp.dot(agg, aw_ref[...], preferred_element_type=F32, precision=lax.Precision.HIGHEST)
                       + ab_ref[...]) + topo
    pre = jax.nn.relu(
        jnp.dot(topo, fw1_ref[...], preferred_element_type=F32, precision=lax.Precision.HIGHEST)
        + jnp.dot(anat, fw2_ref[...], preferred_element_type=F32, precision=lax.Precision.HIGHEST)
        + fb_ref[...])
    mu = jnp.mean(pre, axis=-1, keepdims=True)
    var = jnp.mean((pre - mu) ** 2, axis=-1, keepdims=True)
    fused = (pre - mu) / jnp.sqrt(var + 1e-5) * lg_ref[...] + lb_ref[...]
    nop = jnp.dot(jax.nn.relu(
        jnp.dot(fused, nw1_ref[...], preferred_element_type=F32, precision=lax.Precision.HIGHEST) + nb1_ref[...]),
        nw2_ref[...], preferred_element_type=F32, precision=lax.Precision.HIGHEST) + nb2_ref[...]
    ncr = jnp.dot(jax.nn.relu(
        jnp.dot(fused, cw1_ref[...], preferred_element_type=F32, precision=lax.Precision.HIGHEST) + cb1_ref[...]),
        cw2_ref[...], preferred_element_type=F32, precision=lax.Precision.HIGHEST) + cb2_ref[...]
    q = jax.nn.sigmoid(jnp.dot(jax.nn.relu(
        jnp.dot(fused, qw1_ref[...], preferred_element_type=F32, precision=lax.Precision.HIGHEST) + qb1_ref[...]),
        qw2_ref[...], preferred_element_type=F32, precision=lax.Precision.HIGHEST) + qb2_ref[...])
    nout_ref[...] = jnp.concatenate([nop, ncr, q], axis=-1)
    for c in range(NC16):
        fchunks_ref[c] = fused[:, c * CW:(c + 1) * CW]


def _tc_fusion(agg_tbl, topo, p):
    cst = lambda shp: pl.BlockSpec(shp, lambda i: tuple(0 for _ in shp))
    return pl.pallas_call(
        _fusion_body,
        grid=(NP // BLK,),
        in_specs=[
            pl.BlockSpec((NC16, BLK, CW), lambda i: (0, i, 0)),
            pl.BlockSpec((BLK, HID), lambda i: (i, 0)),
            cst((HID, HID)), cst((1, HID)),
            cst((HID, HID)), cst((HID, HID)), cst((1, HID)),
            cst((1, HID)), cst((1, HID)),
            cst((HID, 64)), cst((1, 64)), cst((64, 3)), cst((1, 3)),
            cst((HID, 64)), cst((1, 64)), cst((64, 7)), cst((1, 7)),
            cst((HID, 32)), cst((1, 32)), cst((32, 1)), cst((1, 1)),
        ],
        out_specs=[
            pl.BlockSpec((BLK, 11), lambda i: (i, 0)),
            pl.BlockSpec((NC16, BLK, CW), lambda i: (0, i, 0)),
        ],
        out_shape=[
            jax.ShapeDtypeStruct((NP, 11), F32),
            jax.ShapeDtypeStruct((NC16, NP, CW), F32),
        ],
    )(agg_tbl, topo,
      p['anat_W'], p['anat_b'].reshape(1, HID),
      p['fuse_W'][:HID], p['fuse_W'][HID:], p['fuse_b'].reshape(1, HID),
      p['ln_g'].reshape(1, HID), p['ln_b'].reshape(1, HID),
      p['nop_W1'], p['nop_b1'].reshape(1, 64), p['nop_W2'], p['nop_b2'].reshape(1, 3),
      p['ncr_W1'], p['ncr_b1'].reshape(1, 64), p['ncr_W2'], p['ncr_b2'].reshape(1, 7),
      p['q_W1'], p['q_b1'].reshape(1, 32), p['q_W2'], p['q_b2'].reshape(1, 1))


def _edge_mlp_body(ef_ref, e0_ref, e1_ref, e2_ref, e3_ref, e4_ref, e5_ref,
                   e6_ref, e7_ref, we_ref, be_ref,
                   w1_ref, b1_ref, w2_ref, b2_ref, out_ref):
    es = [e0_ref, e1_ref, e2_ref, e3_ref, e4_ref, e5_ref, e6_ref, e7_ref]
    ef = ef_ref[...]
    acc = jnp.zeros((ef.shape[0], 32), F32)
    for c in range(NC16):
        sl = slice(c * CW, (c + 1) * CW)
        eh_c = jax.nn.relu(
            jnp.dot(ef, we_ref[...][:, sl], preferred_element_type=F32, precision=lax.Precision.HIGHEST)
            + be_ref[...][:, sl])
        acc = acc + jnp.dot(eh_c + es[c][...], w1_ref[...][sl, :],
                            preferred_element_type=F32, precision=lax.Precision.HIGHEST)
    h1 = jax.nn.relu(acc + b1_ref[...])
    out_ref[...] = jnp.dot(h1, w2_ref[...], preferred_element_type=F32, precision=lax.Precision.HIGHEST) + b2_ref[...]


def _tc_edge_mlp(ef, esumf, p):
    cst = lambda shp: pl.BlockSpec(shp, lambda i: tuple(0 for _ in shp))
    nb = EP // EBLK
    espec = lambda c: pl.BlockSpec((EBLK, CW), lambda i, c=c: (c * nb + i, 0))
    return pl.pallas_call(
        _edge_mlp_body,
        grid=(nb,),
        in_specs=[
            pl.BlockSpec((EBLK, D_EDGE), lambda i: (i, 0)),
            espec(0), espec(1), espec(2), espec(3),
            espec(4), espec(5), espec(6), espec(7),
            cst((D_EDGE, HID)), cst((1, HID)),
            cst((HID, 32)), cst((1, 32)), cst((32, 3)), cst((1, 3)),
        ],
        out_specs=pl.BlockSpec((EBLK, 3), lambda i: (i, 0)),
        out_shape=jax.ShapeDtypeStruct((EP, 3), F32),
    )(ef, esumf, esumf, esumf, esumf, esumf, esumf, esumf, esumf,
      p['enc_edge_W'], p['enc_edge_b'].reshape(1, HID),
      p['eop_W1'], p['eop_b1'].reshape(1, 32), p['eop_W2'], p['eop_b2'].reshape(1, 3))


def _sc_edge_logits(n_heads, act):
    tph = 32 // n_heads
    rpt = EROWS // tph
    chr_ = 8
    nch = rpt // chr_

    @functools.partial(
        pl.kernel,
        compiler_params=pltpu.CompilerParams(
            needs_layout_passes=False, use_tc_tiling_on_sc=False),
        out_type=jax.ShapeDtypeStruct((n_heads, EROWS, 128), F32),
        mesh=_MESH,
        scratch_types=[
            pltpu.VMEM((NP,), F32), pltpu.VMEM((NP,), F32),
            pltpu.VMEM((16, 128), I32), pltpu.VMEM((16, 128), I32),
            pltpu.VMEM((16, 128), F32),
        ],
    )
    def k(sa_hbm, sb_hbm, srcm_hbm, dstm_hbm, e_hbm, cola, colb, sb2, db2, eb2):
        core = lax.axis_index("c")
        sub = lax.axis_index("s")
        if n_heads == 4:
            head = core * 2 + sub // 8
            slot = core * 0 + sub % 8
        else:
            head = core * 0 + sub * 0
            slot = core * 16 + sub
        pltpu.sync_copy(sa_hbm.at[head], cola)
        pltpu.sync_copy(sb_hbm.at[head], colb)
        row0 = slot * rpt

        def chunk(ci, _):
            r = row0 + ci * chr_
            pltpu.sync_copy(srcm_hbm.at[pl.ds(r, chr_), :],
                            sb2.at[pl.ds(0, chr_), :])
            pltpu.sync_copy(dstm_hbm.at[pl.ds(r, chr_), :],
                            db2.at[pl.ds(0, chr_), :])

            def row(j, _):
                for l in range(8):
                    isrc = sb2[j, pl.ds(l * 16, 16)]
                    idst = db2[j, pl.ds(l * 16, 16)]
                    a = plsc.load_gather(cola, [isrc])
                    b = plsc.load_gather(colb, [idst])
                    v = a + b
                    if act == "lrelu":
                        r_ = jnp.where(v >= 0, v, 0.2 * v)
                    else:
                        r_ = 1.0 / (1.0 + jnp.exp(-v))
                    eb2[j, pl.ds(l * 16, 16)] = r_
                return 0

            lax.fori_loop(0, chr_, row, 0)
            pltpu.sync_copy(eb2.at[pl.ds(0, chr_), :],
                            e_hbm.at[head, pl.ds(r, chr_), :])
            return 0

        lax.fori_loop(0, nch, chunk, 0)

    return k


def _sc_segment_max():
    tph = 8
    rpt = EROWS // tph
    nch = rpt // 16

    @functools.partial(
        pl.kernel,
        compiler_params=pltpu.CompilerParams(
            needs_layout_passes=False, use_tc_tiling_on_sc=False),
        out_type=jax.ShapeDtypeStruct((HEADS, NP), F32),
        mesh=_MESH,
        scratch_types=[
            pltpu.VMEM((NP,), F32),
            pltpu.VMEM((16, 128), I32), pltpu.VMEM((16, 128), F32),
            pltpu.VMEM((3200,), F32), pltpu.VMEM((3200,), F32),
            pltpu.VMEM_SHARED((16, NP), F32),
        ],
    )
    def k(e_hbm, dstm_hbm, m_hbm, macc, db2, eb2, cbuf, tbuf, msh):
        core = lax.axis_index("c")
        sub = lax.axis_index("s")
        head = core * 2 + sub // 8
        slot = sub % 8
        neg = jnp.full((16,), NEG_BIG, F32)

        def ini(i, _):
            macc[pl.ds(i * 16, 16)] = neg
            return 0
        lax.fori_loop(0, NP // 16, ini, 0)

        iota = lax.iota(I32, 16)
        row0 = slot * rpt

        def chunk(ci, _):
            r = row0 + ci * 16
            pltpu.sync_copy(dstm_hbm.at[pl.ds(r, 16), :], db2)
            pltpu.sync_copy(e_hbm.at[head, pl.ds(r, 16), :], eb2)

            def row(j, _):
                for l in range(8):
                    idx = db2[j, pl.ds(l * 16, 16)]
                    ev = eb2[j, pl.ds(l * 16, 16)]
                    acc = ev
                    for kk in range(1, 16):
                        perm = (iota + kk) & 15
                        oi = _lane_gather(idx, perm)
                        ov = _lane_gather(ev, perm)
                        acc = jnp.where(oi == idx, jnp.maximum(acc, ov), acc)
                    old = plsc.load_gather(macc, [idx])
                    plsc.store_scatter(macc, [idx], jnp.maximum(old, acc))
                return 0

            lax.fori_loop(0, 16, row, 0)
            return 0

        lax.fori_loop(0, nch, chunk, 0)
        pltpu.sync_copy(macc, msh.at[sub])
        plsc.subcore_barrier()
        for g in range(2):
            pltpu.sync_copy(msh.at[g * 8, pl.ds(sub * 3200, 3200)], cbuf)
            for rcp in range(1, 8):
                pltpu.sync_copy(msh.at[g * 8 + rcp, pl.ds(sub * 3200, 3200)], tbuf)

                def mx(i, _):
                    cbuf[pl.ds(i * 16, 16)] = jnp.maximum(
                        cbuf[pl.ds(i * 16, 16)], tbuf[pl.ds(i * 16, 16)])
                    return 0
                lax.fori_loop(0, 200, mx, 0)
            pltpu.sync_copy(cbuf, m_hbm.at[core * 2 + g, pl.ds(sub * 3200, 3200)])
        plsc.subcore_barrier()

    return k


def _sc_exp_den():
    tph = 8
    rpt = EROWS // tph
    nch = rpt // 16

    @functools.partial(
        pl.kernel,
        compiler_params=pltpu.CompilerParams(
            needs_layout_passes=False, use_tc_tiling_on_sc=False),
        out_type=jax.ShapeDtypeStruct((HEADS, EROWS, 128), F32),
        mesh=_MESH,
        scratch_types=[
            pltpu.VMEM((NP,), F32),
            pltpu.VMEM((16, 128), I32), pltpu.VMEM((16, 128), I32),
            pltpu.VMEM((16, 128), F32), pltpu.VMEM((16, 128), F32),
            pltpu.VMEM((3200,), F32),
            pltpu.VMEM_SHARED((2 * NP,), F32),
        ],
    )
    def k(e_hbm, m_hbm, dstm_hbm, ex_hbm,
          mcol, db2, ab2, eb2, xb2, zbuf, densh):
        core = lax.axis_index("c")
        sub = lax.axis_index("s")
        head = core * 2 + sub // 8
        lhead = sub // 8
        slot = sub % 8
        pltpu.sync_copy(m_hbm.at[head], mcol)

        def z(i, _):
            zbuf[pl.ds(i * 16, 16)] = jnp.zeros((16,), F32)
            return 0
        lax.fori_loop(0, 200, z, 0)
        pltpu.sync_copy(zbuf, densh.at[pl.ds(lhead * NP + slot * 6400, 3200)])
        pltpu.sync_copy(zbuf, densh.at[pl.ds(lhead * NP + slot * 6400 + 3200, 3200)])
        plsc.subcore_barrier()

        row0 = slot * rpt
        base = lhead * NP

        def chunk(ci, _):
            r = row0 + ci * 16
            pltpu.sync_copy(dstm_hbm.at[pl.ds(r, 16), :], db2)
            pltpu.sync_copy(e_hbm.at[head, pl.ds(r, 16), :], eb2)

            def row(j, _):
                for l in range(8):
                    idx = db2[j, pl.ds(l * 16, 16)]
                    ev = eb2[j, pl.ds(l * 16, 16)]
                    mv = plsc.load_gather(mcol, [idx])
                    xb2[j, pl.ds(l * 16, 16)] = jnp.exp(ev - mv)
                    ab2[j, pl.ds(l * 16, 16)] = idx + base
                return 0

            lax.fori_loop(0, 16, row, 0)
            pltpu.sync_copy(xb2, ex_hbm.at[head, pl.ds(r, 16), :])

            def srow(j, _):
                pltpu.sync_copy(xb2.at[j], densh.at[ab2.at[j]], add=True)
                return 0
            lax.fori_loop(0, 16, srow, 0)
            return 0

        lax.fori_loop(0, nch, chunk, 0)
        plsc.subcore_barrier()
        pltpu.sync_copy(densh.at[pl.ds(lhead * NP, NP)], mcol)

        def chunk2(ci, _):
            r = row0 + ci * 16
            pltpu.sync_copy(dstm_hbm.at[pl.ds(r, 16), :], db2)
            pltpu.sync_copy(ex_hbm.at[head, pl.ds(r, 16), :], xb2)

            def row2(j, _):
                for l in range(8):
                    idx = db2[j, pl.ds(l * 16, 16)]
                    xv = xb2[j, pl.ds(l * 16, 16)]
                    dv = plsc.load_gather(mcol, [idx])
                    xb2[j, pl.ds(l * 16, 16)] = xv / (dv + 1e-9)
                return 0

            lax.fori_loop(0, 16, row2, 0)
            pltpu.sync_copy(xb2, ex_hbm.at[head, pl.ds(r, 16), :])
            return 0

        lax.fori_loop(0, nch, chunk2, 0)
        plsc.subcore_barrier()

    return k


def _sc_weighted_scatter(head_w):
    rpt = EROWS // 16
    nch = rpt // 16

    scratch = [
        pltpu.VMEM((16, 128), I32), pltpu.VMEM((16, 128), I32),
        pltpu.VMEM((16, 128), I32), pltpu.VMEM((16, 128), F32),
        pltpu.VMEM((2048, CW), F32),
        pltpu.VMEM_SHARED((NP, CW), F32),
        pltpu.SemaphoreType.DMA,
    ]

    @functools.partial(
        pl.kernel,
        compiler_params=pltpu.CompilerParams(
            needs_layout_passes=False, use_tc_tiling_on_sc=False),
        out_type=jax.ShapeDtypeStruct((NC16 * NP, CW), F32),
        mesh=_MESH,
        scratch_types=scratch,
    )
    def k(tblf_hbm, w_hbm, srcm_hbm, dstm_hbm, zer_hbm, out_hbm, *scr):
        sb2, db2, ab2, wb2, rows, acc, sem = scr
        core = lax.axis_index("c")
        sub = lax.axis_index("s")
        row0 = sub * rpt

        for li in range(4):
            c = core * 4 + li
            wrow = (c >> 1) if head_w else c * 0
            pltpu.sync_copy(zer_hbm.at[pl.ds(sub * 3200, 3200), :],
                            acc.at[pl.ds(sub * 3200, 3200), :])
            plsc.subcore_barrier()

            def chunk(ci, c=c, wrow=wrow):
                r = row0 + ci * 16
                pltpu.sync_copy(srcm_hbm.at[pl.ds(r, 16), :], sb2)
                pltpu.sync_copy(dstm_hbm.at[pl.ds(r, 16), :], db2)
                pltpu.sync_copy(w_hbm.at[wrow, pl.ds(r, 16), :], wb2)

                def adj(j, _):
                    for l in range(8):
                        ab2[j, pl.ds(l * 16, 16)] = (
                            sb2[j, pl.ds(l * 16, 16)] + c * NP)
                    return 0
                lax.fori_loop(0, 16, adj, 0)

                def gat(j, _):
                    pltpu.async_copy(
                        tblf_hbm.at[ab2.at[j]],
                        rows.at[pl.ds(j * 128, 128), :], sem).wait()
                    return 0
                lax.fori_loop(0, 16, gat, 0)

                def scale(j, _):
                    for l in range(8):
                        coef = wb2[j, pl.ds(l * 16, 16)]
                        for jj in range(16):
                            e_in_row = l * 16 + jj
                            cj = coef[jj]
                            r0 = rows[j * 128 + e_in_row, pl.ds(0, 16)]
                            rows[j * 128 + e_in_row, pl.ds(0, 16)] = r0 * cj
                    return 0
                lax.fori_loop(0, 16, scale, 0)

                def sct(j, _):
                    pltpu.sync_copy(rows.at[pl.ds(j * 128, 128), :],
                                    acc.at[db2.at[j]], add=True)
                    return 0
                lax.fori_loop(0, 16, sct, 0)

            def chunk_body(ci, _):
                chunk(ci)
                return 0
            lax.fori_loop(0, nch, chunk_body, 0)
            plsc.subcore_barrier()
            pltpu.sync_copy(
                acc.at[pl.ds(sub * 3200, 3200), :],
                out_hbm.at[pl.ds(c * NP + sub * 3200, 3200), :])
            plsc.subcore_barrier()

    return k


def _sc_pair_sum():
    rpt = EROWS // 16
    nch = rpt // 16

    @functools.partial(
        pl.kernel,
        compiler_params=pltpu.CompilerParams(
            needs_layout_passes=False, use_tc_tiling_on_sc=False),
        out_type=jax.ShapeDtypeStruct((NC16 * EP, CW), F32),
        mesh=_MESH,
        scratch_types=[
            pltpu.VMEM((16, 128), I32), pltpu.VMEM((16, 128), I32),
            pltpu.VMEM((2048, CW), F32),
            pltpu.VMEM_SHARED((NP, CW), F32),
            pltpu.SemaphoreType.DMA,
        ],
    )
    def k(tblf_hbm, srcm_hbm, dstm_hbm, out_hbm, sb2, db2, rows, sh, sem):
        core = lax.axis_index("c")
        sub = lax.axis_index("s")
        row0 = sub * rpt

        for li in range(4):
            c = core * 4 + li
            pltpu.sync_copy(
                tblf_hbm.at[pl.ds(c * NP + sub * 3200, 3200), :],
                sh.at[pl.ds(sub * 3200, 3200), :])
            plsc.subcore_barrier()

            def chunk(ci, c=c):
                r = row0 + ci * 16
                pltpu.sync_copy(srcm_hbm.at[pl.ds(r, 16), :], sb2)
                pltpu.sync_copy(dstm_hbm.at[pl.ds(r, 16), :], db2)

                def gat(j, _):
                    pltpu.async_copy(sh.at[sb2.at[j]],
                                     rows.at[pl.ds(j * 128, 128), :], sem).wait()
                    pltpu.async_copy(sh.at[db2.at[j]],
                                     rows.at[pl.ds(j * 128, 128), :], sem,
                                     add=True).wait()
                    return 0
                lax.fori_loop(0, 16, gat, 0)
                pltpu.sync_copy(rows,
                                out_hbm.at[pl.ds(c * EP + r * 128, 2048), :])

            def chunk_body(ci, _):
                chunk(ci)
                return 0
            lax.fori_loop(0, nch, chunk_body, 0)
            plsc.subcore_barrier()

    return k


def kernel(node_features, edge_features, node_positions, node_radii,
           edge_index, node_types, params):
    p = params

    nf = jnp.pad(node_features.astype(F32), ((0, NP - N_NODES), (0, 0)))
    types = jnp.pad(node_types.astype(I32), (0, NP - N_NODES)).reshape(NP, 1)
    pos = jnp.pad(node_positions.astype(F32), ((0, NP - N_NODES), (0, 0)))
    rad = jnp.pad(node_radii.astype(F32), (0, NP - N_NODES)).reshape(NP, 1)

    ei = edge_index.astype(I32)
    npad = EP - N_EDGES
    fill = N_NODES + (jnp.arange(npad, dtype=I32) % (NP - N_NODES))
    src = jnp.concatenate([ei[0], fill])
    dst = jnp.concatenate([ei[1], fill])
    srcm = src.reshape(EROWS, 128)
    dstm = dst.reshape(EROWS, 128)
    ef = jnp.pad(edge_features.astype(F32), ((0, npad), (0, 0)))

    mask = jnp.repeat(jnp.eye(HEADS, dtype=F32), HDIM, axis=0)
    zeros_tbl = jnp.zeros((NP, CW), F32)

    x = _tc_encode(nf, types, p['enc_node_W'], p['enc_node_b'].reshape(1, HID),
                   p['type_emb'])

    e_logits4 = _sc_edge_logits(4, "lrelu")
    e_logits1 = _sc_edge_logits(1, "sigmoid")
    seg_max = _sc_segment_max()
    exp_den = _sc_exp_den()
    wsc_gat = _sc_weighted_scatter(True)
    wsc_anat = _sc_weighted_scatter(False)
    pair_sum = _sc_pair_sum()

    topo = x
    topo_chunks = None
    for i in range(3):
        A8 = jnp.concatenate(
            [mask * p['gat%d_asrc' % i].reshape(-1, 1),
             mask * p['gat%d_adst' % i].reshape(-1, 1)], axis=1)
        h_tbl, s8 = _tc_layer_proj(topo, p['gat%d_W' % i], A8)
        sT = jnp.transpose(s8)
        e = e_logits4(sT[:4], sT[4:], srcm, dstm)
        m = seg_max(e, dstm)
        coef = exp_den(e, m, dstm)
        outf = wsc_gat(h_tbl.reshape(NC16 * NP, CW), coef,
                       srcm, dstm, zeros_tbl)
        res = _tc_elu_res(outf.reshape(NC16, NP, CW), topo,
                          with_chunks=(i == 2))
        topo = res[0]
        if i == 2:
            topo_chunks = res[1]

    ua, ub = _tc_geo(pos, rad, p['geo_W'].reshape(1, 4),
                     p['geo_b'].reshape(1, 1))
    gate = e_logits1(jnp.transpose(ua), jnp.transpose(ub), srcm, dstm)
    aggf = wsc_anat(topo_chunks.reshape(NC16 * NP, CW), gate,
                    srcm, dstm, zeros_tbl)
    nout, fchunks = _tc_fusion(aggf.reshape(NC16, NP, CW), topo, p)

    esumf = pair_sum(fchunks.reshape(NC16 * NP, CW), srcm, dstm)
    edge_op = _tc_edge_mlp(ef, esumf, p)

    return nout[:N_NODES], edge_op[:N_EDGES]


# speedup vs baseline: 17.8158x; 1.2494x over previous
import functools

import jax
import jax.numpy as jnp
from jax import lax
from jax.experimental import pallas as pl
from jax.experimental.pallas import tpu as pltpu
from jax.experimental.pallas import tpu_sc as plsc

F32 = jnp.float32
I32 = jnp.int32

N_NODES = 50000
NP = 51200
N_EDGES = 800000
EP = 819200
EROWS = EP // 128
D_NODE = 16
D_EDGE = 8
HID = 128
HEADS = 4
HDIM = 32
NC16 = 8
CW = 16

BLK = 1600
EBLK = 2048

_MESH = plsc.VectorSubcoreMesh(core_axis_name="c", subcore_axis_name="s")
NEG_BIG = -3.0e38

_GDN = lax.GatherDimensionNumbers(
    offset_dims=(), collapsed_slice_dims=(0,), start_index_map=(0,))


def _lane_gather(x, perm):
    return lax.gather(x, perm[:, None], _GDN, (1,),
                      mode=lax.GatherScatterMode.PROMISE_IN_BOUNDS)


def _enc_body(nf_ref, types_ref, w_ref, b_ref, temb_ref, out_ref):
    x = jnp.dot(nf_ref[...], w_ref[...], preferred_element_type=F32, precision=lax.Precision.HIGHEST)
    x = x + b_ref[...]
    t = types_ref[...]
    for k in range(3):
        x = x + jnp.where(t == k, temb_ref[k, :][None, :], 0.0)
    out_ref[...] = x


def _tc_encode(nf, types, W, b, temb):
    return pl.pallas_call(
        _enc_body,
        grid=(NP // BLK,),
        in_specs=[
            pl.BlockSpec((BLK, D_NODE), lambda i: (i, 0)),
            pl.BlockSpec((BLK, 1), lambda i: (i, 0)),
            pl.BlockSpec((D_NODE, HID), lambda i: (0, 0)),
            pl.BlockSpec((1, HID), lambda i: (0, 0)),
            pl.BlockSpec((3, HID), lambda i: (0, 0)),
        ],
        out_specs=pl.BlockSpec((BLK, HID), lambda i: (i, 0)),
        out_shape=jax.ShapeDtypeStruct((NP, HID), F32),
    )(nf, types, W, b, temb)


def _proj_body(x_ref, w_ref, a8_ref, h_ref, s_ref):
    h = jnp.dot(x_ref[...], w_ref[...], preferred_element_type=F32, precision=lax.Precision.HIGHEST)
    for c in range(NC16):
        h_ref[c] = h[:, c * CW:(c + 1) * CW]
    s_ref[...] = jnp.dot(h, a8_ref[...], preferred_element_type=F32, precision=lax.Precision.HIGHEST)


def _tc_layer_proj(x, W, A8):
    return pl.pallas_call(
        _proj_body,
        grid=(NP // BLK,),
        in_specs=[
            pl.BlockSpec((BLK, HID), lambda i: (i, 0)),
            pl.BlockSpec((HID, HID), lambda i: (0, 0)),
            pl.BlockSpec((HID, 8), lambda i: (0, 0)),
        ],
        out_specs=[
            pl.BlockSpec((NC16, BLK, CW), lambda i: (0, i, 0)),
            pl.BlockSpec((BLK, 8), lambda i: (i, 0)),
        ],
        out_shape=[
            jax.ShapeDtypeStruct((NC16, NP, CW), F32),
            jax.ShapeDtypeStruct((NP, 8), F32),
        ],
    )(x, W, A8)


def _elu_res_body(o_ref, t_ref, out_ref, chunks_ref):
    o = jnp.concatenate([o_ref[c] for c in range(NC16)], axis=-1)
    r = jnp.where(o > 0, o, jnp.exp(jnp.minimum(o, 0.0)) - 1.0) + t_ref[...]
    out_ref[...] = r
    if chunks_ref is not None:
        for c in range(NC16):
            chunks_ref[c] = r[:, c * CW:(c + 1) * CW]


def _tc_elu_res(out_tbl, topo, with_chunks):
    body = _elu_res_body if with_chunks else (
        lambda o_ref, t_ref, out_ref: _elu_res_body(o_ref, t_ref, out_ref, None))
    out_specs = [pl.BlockSpec((BLK, HID), lambda i: (i, 0))]
    out_shape = [jax.ShapeDtypeStruct((NP, HID), F32)]
    if with_chunks:
        out_specs.append(pl.BlockSpec((NC16, BLK, CW), lambda i: (0, i, 0)))
        out_shape.append(jax.ShapeDtypeStruct((NC16, NP, CW), F32))
    return pl.pallas_call(
        body,
        grid=(NP // BLK,),
        in_specs=[
            pl.BlockSpec((NC16, BLK, CW), lambda i: (0, i, 0)),
            pl.BlockSpec((BLK, HID), lambda i: (i, 0)),
        ],
        out_specs=out_specs,
        out_shape=out_shape,
    )(out_tbl, topo)


def _geo_body(pos_ref, rad_ref, gw_ref, gb_ref, ua_ref, ub_ref):
    u = jnp.sum(pos_ref[...] * gw_ref[0:1, 0:3], axis=1, keepdims=True)
    u = u + rad_ref[...] * gw_ref[0:1, 3:4]
    ua_ref[...] = u + gb_ref[...]
    ub_ref[...] = -u


def _tc_geo(pos, rad, gw_row, gb):
    return pl.pallas_call(
        _geo_body,
        grid=(NP // BLK,),
        in_specs=[
            pl.BlockSpec((BLK, 3), lambda i: (i, 0)),
            pl.BlockSpec((BLK, 1), lambda i: (i, 0)),
            pl.BlockSpec((1, 4), lambda i: (0, 0)),
            pl.BlockSpec((1, 1), lambda i: (0, 0)),
        ],
        out_specs=[
            pl.BlockSpec((BLK, 1), lambda i: (i, 0)),
            pl.BlockSpec((BLK, 1), lambda i: (i, 0)),
        ],
        out_shape=[
            jax.ShapeDtypeStruct((NP, 1), F32),
            jax.ShapeDtypeStruct((NP, 1), F32),
        ],
    )(pos, rad, gw_row, gb)


def _fusion_body(agg_ref, topo_ref, aw_ref, ab_ref, fw1_ref, fw2_ref, fb_ref,
                 lg_ref, lb_ref, nw1_ref, nb1_ref, nw2_ref, nb2_ref,
                 cw1_ref, cb1_ref, cw2_ref, cb2_ref, qw1_ref, qb1_ref,
                 qw2_ref, qb2_ref, nout_ref, fchunks_ref):
    agg = jnp.concatenate([agg_ref[c] for c in range(NC16)], axis=-1)
    topo = topo_ref[...]
    anat = jax.nn.relu(jnp.dot(agg, aw_ref[...], preferred_element_type=F32, precision=lax.Precision.HIGHEST)
                       + ab_ref[...]) + topo
    pre = jax.nn.relu(
        jnp.dot(topo, fw1_ref[...], preferred_element_type=F32, precision=lax.Precision.HIGHEST)
        + jnp.dot(anat, fw2_ref[...], preferred_element_type=F32, precision=lax.Precision.HIGHEST)
        + fb_ref[...])
    mu = jnp.mean(pre, axis=-1, keepdims=True)
    var = jnp.mean((pre - mu) ** 2, axis=-1, keepdims=True)
    fused = (pre - mu) / jnp.sqrt(var + 1e-5) * lg_ref[...] + lb_ref[...]
    nop = jnp.dot(jax.nn.relu(
        jnp.dot(fused, nw1_ref[...], preferred_element_type=F32, precision=lax.Precision.HIGHEST) + nb1_ref[...]),
        nw2_ref[...], preferred_element_type=F32, precision=lax.Precision.HIGHEST) + nb2_ref[...]
    ncr = jnp.dot(jax.nn.relu(
        jnp.dot(fused, cw1_ref[...], preferred_element_type=F32, precision=lax.Precision.HIGHEST) + cb1_ref[...]),
        cw2_ref[...], preferred_element_type=F32, precision=lax.Precision.HIGHEST) + cb2_ref[...]
    q = jax.nn.sigmoid(jnp.dot(jax.nn.relu(
        jnp.dot(fused, qw1_ref[...], preferred_element_type=F32, precision=lax.Precision.HIGHEST) + qb1_ref[...]),
        qw2_ref[...], preferred_element_type=F32, precision=lax.Precision.HIGHEST) + qb2_ref[...])
    nout_ref[...] = jnp.concatenate([nop, ncr, q], axis=-1)
    for c in range(NC16):
        fchunks_ref[c] = fused[:, c * CW:(c + 1) * CW]


def _tc_fusion(agg_tbl, topo, p):
    cst = lambda shp: pl.BlockSpec(shp, lambda i: tuple(0 for _ in shp))
    return pl.pallas_call(
        _fusion_body,
        grid=(NP // BLK,),
        in_specs=[
            pl.BlockSpec((NC16, BLK, CW), lambda i: (0, i, 0)),
            pl.BlockSpec((BLK, HID), lambda i: (i, 0)),
            cst((HID, HID)), cst((1, HID)),
            cst((HID, HID)), cst((HID, HID)), cst((1, HID)),
            cst((1, HID)), cst((1, HID)),
            cst((HID, 64)), cst((1, 64)), cst((64, 3)), cst((1, 3)),
            cst((HID, 64)), cst((1, 64)), cst((64, 7)), cst((1, 7)),
            cst((HID, 32)), cst((1, 32)), cst((32, 1)), cst((1, 1)),
        ],
        out_specs=[
            pl.BlockSpec((BLK, 11), lambda i: (i, 0)),
            pl.BlockSpec((NC16, BLK, CW), lambda i: (0, i, 0)),
        ],
        out_shape=[
            jax.ShapeDtypeStruct((NP, 11), F32),
            jax.ShapeDtypeStruct((NC16, NP, CW), F32),
        ],
    )(agg_tbl, topo,
      p['anat_W'], p['anat_b'].reshape(1, HID),
      p['fuse_W'][:HID], p['fuse_W'][HID:], p['fuse_b'].reshape(1, HID),
      p['ln_g'].reshape(1, HID), p['ln_b'].reshape(1, HID),
      p['nop_W1'], p['nop_b1'].reshape(1, 64), p['nop_W2'], p['nop_b2'].reshape(1, 3),
      p['ncr_W1'], p['ncr_b1'].reshape(1, 64), p['ncr_W2'], p['ncr_b2'].reshape(1, 7),
      p['q_W1'], p['q_b1'].reshape(1, 32), p['q_W2'], p['q_b2'].reshape(1, 1))


def _edge_mlp_body(ef_ref, e0_ref, e1_ref, e2_ref, e3_ref, e4_ref, e5_ref,
                   e6_ref, e7_ref, we_ref, be_ref,
                   w1_ref, b1_ref, w2_ref, b2_ref, out_ref):
    es = [e0_ref, e1_ref, e2_ref, e3_ref, e4_ref, e5_ref, e6_ref, e7_ref]
    ef = ef_ref[...]
    acc = jnp.zeros((ef.shape[0], 32), F32)
    for c in range(NC16):
        sl = slice(c * CW, (c + 1) * CW)
        eh_c = jax.nn.relu(
            jnp.dot(ef, we_ref[...][:, sl], preferred_element_type=F32, precision=lax.Precision.HIGHEST)
            + be_ref[...][:, sl])
        acc = acc + jnp.dot(eh_c + es[c][...], w1_ref[...][sl, :],
                            preferred_element_type=F32, precision=lax.Precision.HIGHEST)
    h1 = jax.nn.relu(acc + b1_ref[...])
    out_ref[...] = jnp.dot(h1, w2_ref[...], preferred_element_type=F32, precision=lax.Precision.HIGHEST) + b2_ref[...]


def _tc_edge_mlp(ef, esumf, p):
    cst = lambda shp: pl.BlockSpec(shp, lambda i: tuple(0 for _ in shp))
    nb = EP // EBLK
    espec = lambda c: pl.BlockSpec((EBLK, CW), lambda i, c=c: (c * nb + i, 0))
    return pl.pallas_call(
        _edge_mlp_body,
        grid=(nb,),
        in_specs=[
            pl.BlockSpec((EBLK, D_EDGE), lambda i: (i, 0)),
            espec(0), espec(1), espec(2), espec(3),
            espec(4), espec(5), espec(6), espec(7),
            cst((D_EDGE, HID)), cst((1, HID)),
            cst((HID, 32)), cst((1, 32)), cst((32, 3)), cst((1, 3)),
        ],
        out_specs=pl.BlockSpec((EBLK, 3), lambda i: (i, 0)),
        out_shape=jax.ShapeDtypeStruct((EP, 3), F32),
    )(ef, esumf, esumf, esumf, esumf, esumf, esumf, esumf, esumf,
      p['enc_edge_W'], p['enc_edge_b'].reshape(1, HID),
      p['eop_W1'], p['eop_b1'].reshape(1, 32), p['eop_W2'], p['eop_b2'].reshape(1, 3))


def _sc_edge_logits(n_heads, act):
    tph = 32 // n_heads
    rpt = EROWS // tph
    chr_ = 40
    nch = rpt // chr_

    @functools.partial(
        pl.kernel,
        compiler_params=pltpu.CompilerParams(
            needs_layout_passes=False, use_tc_tiling_on_sc=False),
        out_type=jax.ShapeDtypeStruct((n_heads, EROWS, 128), F32),
        mesh=_MESH,
        scratch_types=[
            pltpu.VMEM((NP,), F32), pltpu.VMEM((NP,), F32),
            pltpu.VMEM((40, 128), I32), pltpu.VMEM((40, 128), I32),
            pltpu.VMEM((40, 128), F32),
        ],
    )
    def k(sa_hbm, sb_hbm, srcm_hbm, dstm_hbm, e_hbm, cola, colb, sb2, db2, eb2):
        core = lax.axis_index("c")
        sub = lax.axis_index("s")
        if n_heads == 4:
            head = core * 2 + sub // 8
            slot = core * 0 + sub % 8
        else:
            head = core * 0 + sub * 0
            slot = core * 16 + sub
        pltpu.sync_copy(sa_hbm.at[head], cola)
        pltpu.sync_copy(sb_hbm.at[head], colb)
        row0 = slot * rpt

        def chunk(ci, _):
            r = row0 + ci * chr_
            pltpu.sync_copy(srcm_hbm.at[pl.ds(r, chr_), :], sb2)
            pltpu.sync_copy(dstm_hbm.at[pl.ds(r, chr_), :], db2)

            def row(j, _):
                for l in range(8):
                    isrc = sb2[j, pl.ds(l * 16, 16)]
                    idst = db2[j, pl.ds(l * 16, 16)]
                    a = plsc.load_gather(cola, [isrc])
                    b = plsc.load_gather(colb, [idst])
                    v = a + b
                    if act == "lrelu":
                        r_ = jnp.where(v >= 0, v, 0.2 * v)
                    else:
                        r_ = 1.0 / (1.0 + jnp.exp(-v))
                    eb2[j, pl.ds(l * 16, 16)] = r_
                return 0

            lax.fori_loop(0, chr_, row, 0)
            pltpu.sync_copy(eb2, e_hbm.at[head, pl.ds(r, chr_), :])
            return 0

        lax.fori_loop(0, nch, chunk, 0)

    return k


def _sc_segment_max():
    tph = 8
    rpt = EROWS // tph
    chr_ = 40
    nch = rpt // chr_

    @functools.partial(
        pl.kernel,
        compiler_params=pltpu.CompilerParams(
            needs_layout_passes=False, use_tc_tiling_on_sc=False),
        out_type=jax.ShapeDtypeStruct((HEADS, NP), F32),
        mesh=_MESH,
        scratch_types=[
            pltpu.VMEM((NP,), F32),
            pltpu.VMEM((40, 128), I32), pltpu.VMEM((40, 128), F32),
            pltpu.VMEM((3200,), F32), pltpu.VMEM((3200,), F32),
            pltpu.VMEM_SHARED((16, NP), F32),
        ],
    )
    def k(e_hbm, dstm_hbm, m_hbm, macc, db2, eb2, cbuf, tbuf, msh):
        core = lax.axis_index("c")
        sub = lax.axis_index("s")
        head = core * 2 + sub // 8
        slot = sub % 8
        neg = jnp.full((16,), NEG_BIG, F32)

        def ini(i, _):
            macc[pl.ds(i * 16, 16)] = neg
            return 0
        lax.fori_loop(0, NP // 16, ini, 0)

        iota = lax.iota(I32, 16)
        row0 = slot * rpt

        def chunk(ci, _):
            r = row0 + ci * chr_
            pltpu.sync_copy(dstm_hbm.at[pl.ds(r, chr_), :], db2)
            pltpu.sync_copy(e_hbm.at[head, pl.ds(r, chr_), :], eb2)

            def row(j, _):
                for l in range(8):
                    idx = db2[j, pl.ds(l * 16, 16)]
                    ev = eb2[j, pl.ds(l * 16, 16)]
                    acc = ev
                    for kk in range(1, 16):
                        perm = (iota + kk) & 15
                        oi = _lane_gather(idx, perm)
                        ov = _lane_gather(ev, perm)
                        acc = jnp.where(oi == idx, jnp.maximum(acc, ov), acc)
                    old = plsc.load_gather(macc, [idx])
                    plsc.store_scatter(macc, [idx], jnp.maximum(old, acc))
                return 0

            lax.fori_loop(0, chr_, row, 0)
            return 0

        lax.fori_loop(0, nch, chunk, 0)
        pltpu.sync_copy(macc, msh.at[sub])
        plsc.subcore_barrier()
        for g in range(2):
            pltpu.sync_copy(msh.at[g * 8, pl.ds(sub * 3200, 3200)], cbuf)
            for rcp in range(1, 8):
                pltpu.sync_copy(msh.at[g * 8 + rcp, pl.ds(sub * 3200, 3200)], tbuf)

                def mx(i, _):
                    cbuf[pl.ds(i * 16, 16)] = jnp.maximum(
                        cbuf[pl.ds(i * 16, 16)], tbuf[pl.ds(i * 16, 16)])
                    return 0
                lax.fori_loop(0, 200, mx, 0)
            pltpu.sync_copy(cbuf, m_hbm.at[core * 2 + g, pl.ds(sub * 3200, 3200)])
        plsc.subcore_barrier()

    return k


def _sc_exp_den():
    tph = 8
    rpt = EROWS // tph
    chr_ = 40
    nch = rpt // chr_

    @functools.partial(
        pl.kernel,
        compiler_params=pltpu.CompilerParams(
            needs_layout_passes=False, use_tc_tiling_on_sc=False),
        out_type=jax.ShapeDtypeStruct((HEADS, EROWS, 128), F32),
        mesh=_MESH,
        scratch_types=[
            pltpu.VMEM((NP,), F32),
            pltpu.VMEM((40, 128), I32), pltpu.VMEM((40, 128), I32),
            pltpu.VMEM((40, 128), F32), pltpu.VMEM((40, 128), F32),
            pltpu.VMEM((3200,), F32),
            pltpu.VMEM_SHARED((2 * NP,), F32),
        ],
    )
    def k(e_hbm, m_hbm, dstm_hbm, ex_hbm,
          mcol, db2, ab2, eb2, xb2, zbuf, densh):
        core = lax.axis_index("c")
        sub = lax.axis_index("s")
        head = core * 2 + sub // 8
        lhead = sub // 8
        slot = sub % 8
        pltpu.sync_copy(m_hbm.at[head], mcol)

        def z(i, _):
            zbuf[pl.ds(i * 16, 16)] = jnp.zeros((16,), F32)
            return 0
        lax.fori_loop(0, 200, z, 0)
        pltpu.sync_copy(zbuf, densh.at[pl.ds(lhead * NP + slot * 6400, 3200)])
        pltpu.sync_copy(zbuf, densh.at[pl.ds(lhead * NP + slot * 6400 + 3200, 3200)])
        plsc.subcore_barrier()

        row0 = slot * rpt
        base = lhead * NP

        def chunk(ci, _):
            r = row0 + ci * chr_
            pltpu.sync_copy(dstm_hbm.at[pl.ds(r, chr_), :], db2)
            pltpu.sync_copy(e_hbm.at[head, pl.ds(r, chr_), :], eb2)

            def row(j, _):
                for l in range(8):
                    idx = db2[j, pl.ds(l * 16, 16)]
                    ev = eb2[j, pl.ds(l * 16, 16)]
                    mv = plsc.load_gather(mcol, [idx])
                    xb2[j, pl.ds(l * 16, 16)] = jnp.exp(ev - mv)
                    ab2[j, pl.ds(l * 16, 16)] = idx + base
                return 0

            lax.fori_loop(0, chr_, row, 0)
            pltpu.sync_copy(xb2, ex_hbm.at[head, pl.ds(r, chr_), :])

            def srow(j, _):
                pltpu.sync_copy(xb2.at[j], densh.at[ab2.at[j]], add=True)
                return 0
            lax.fori_loop(0, chr_, srow, 0)
            return 0

        lax.fori_loop(0, nch, chunk, 0)
        plsc.subcore_barrier()
        pltpu.sync_copy(densh.at[pl.ds(lhead * NP, NP)], mcol)

        def chunk2(ci, _):
            r = row0 + ci * chr_
            pltpu.sync_copy(dstm_hbm.at[pl.ds(r, chr_), :], db2)
            pltpu.sync_copy(ex_hbm.at[head, pl.ds(r, chr_), :], xb2)

            def row2(j, _):
                for l in range(8):
                    idx = db2[j, pl.ds(l * 16, 16)]
                    xv = xb2[j, pl.ds(l * 16, 16)]
                    dv = plsc.load_gather(mcol, [idx])
                    xb2[j, pl.ds(l * 16, 16)] = xv / (dv + 1e-9)
                return 0

            lax.fori_loop(0, chr_, row2, 0)
            pltpu.sync_copy(xb2, ex_hbm.at[head, pl.ds(r, chr_), :])
            return 0

        lax.fori_loop(0, nch, chunk2, 0)
        plsc.subcore_barrier()

    return k


def _sc_weighted_scatter(head_w):
    rpt = EROWS // 16
    nch = rpt // 16

    scratch = [
        pltpu.VMEM((16, 128), I32), pltpu.VMEM((16, 128), I32),
        pltpu.VMEM((16, 128), I32), pltpu.VMEM((16, 128), F32),
        pltpu.VMEM((2048, CW), F32),
        pltpu.VMEM_SHARED((NP, CW), F32),
        pltpu.SemaphoreType.DMA,
    ]

    @functools.partial(
        pl.kernel,
        compiler_params=pltpu.CompilerParams(
            needs_layout_passes=False, use_tc_tiling_on_sc=False),
        out_type=jax.ShapeDtypeStruct((NC16 * NP, CW), F32),
        mesh=_MESH,
        scratch_types=scratch,
    )
    def k(tblf_hbm, w_hbm, srcm_hbm, dstm_hbm, zer_hbm, out_hbm, *scr):
        sb2, db2, ab2, wb2, rows, acc, sem = scr
        core = lax.axis_index("c")
        sub = lax.axis_index("s")
        row0 = sub * rpt

        for li in range(4):
            c = core * 4 + li
            wrow = (c >> 1) if head_w else c * 0
            pltpu.sync_copy(zer_hbm.at[pl.ds(sub * 3200, 3200), :],
                            acc.at[pl.ds(sub * 3200, 3200), :])
            plsc.subcore_barrier()

            def chunk(ci, c=c, wrow=wrow):
                r = row0 + ci * 16
                pltpu.sync_copy(srcm_hbm.at[pl.ds(r, 16), :], sb2)
                pltpu.sync_copy(dstm_hbm.at[pl.ds(r, 16), :], db2)
                pltpu.sync_copy(w_hbm.at[wrow, pl.ds(r, 16), :], wb2)

                def adj(j, _):
                    for l in range(8):
                        ab2[j, pl.ds(l * 16, 16)] = (
                            sb2[j, pl.ds(l * 16, 16)] + c * NP)
                    return 0
                lax.fori_loop(0, 16, adj, 0)

                for j in range(16):
                    pltpu.async_copy(tblf_hbm.at[ab2.at[j]],
                                     rows.at[pl.ds(j * 128, 128), :], sem)
                for j in range(16):
                    pltpu.make_async_copy(
                        tblf_hbm.at[ab2.at[j]],
                        rows.at[pl.ds(j * 128, 128), :], sem).wait()

                def scale(j, _):
                    for l in range(8):
                        coef = wb2[j, pl.ds(l * 16, 16)]
                        for jj in range(16):
                            e_in_row = l * 16 + jj
                            cj = coef[jj]
                            r0 = rows[j * 128 + e_in_row, pl.ds(0, 16)]
                            rows[j * 128 + e_in_row, pl.ds(0, 16)] = r0 * cj
                    return 0
                lax.fori_loop(0, 16, scale, 0)

                def sct(j, _):
                    pltpu.sync_copy(rows.at[pl.ds(j * 128, 128), :],
                                    acc.at[db2.at[j]], add=True)
                    return 0
                lax.fori_loop(0, 16, sct, 0)

            def chunk_body(ci, _):
                chunk(ci)
                return 0
            lax.fori_loop(0, nch, chunk_body, 0)
            plsc.subcore_barrier()
            pltpu.sync_copy(
                acc.at[pl.ds(sub * 3200, 3200), :],
                out_hbm.at[pl.ds(c * NP + sub * 3200, 3200), :])
            plsc.subcore_barrier()

    return k


def _sc_pair_sum():
    rpt = EROWS // 16
    nch = rpt // 16

    @functools.partial(
        pl.kernel,
        compiler_params=pltpu.CompilerParams(
            needs_layout_passes=False, use_tc_tiling_on_sc=False),
        out_type=jax.ShapeDtypeStruct((NC16 * EP, CW), F32),
        mesh=_MESH,
        scratch_types=[
            pltpu.VMEM((16, 128), I32), pltpu.VMEM((16, 128), I32),
            pltpu.VMEM((2048, CW), F32),
            pltpu.VMEM_SHARED((NP, CW), F32),
            pltpu.SemaphoreType.DMA,
        ],
    )
    def k(tblf_hbm, srcm_hbm, dstm_hbm, out_hbm, sb2, db2, rows, sh, sem):
        core = lax.axis_index("c")
        sub = lax.axis_index("s")
        row0 = sub * rpt

        for li in range(4):
            c = core * 4 + li
            pltpu.sync_copy(
                tblf_hbm.at[pl.ds(c * NP + sub * 3200, 3200), :],
                sh.at[pl.ds(sub * 3200, 3200), :])
            plsc.subcore_barrier()

            def chunk(ci, c=c):
                r = row0 + ci * 16
                pltpu.sync_copy(srcm_hbm.at[pl.ds(r, 16), :], sb2)
                pltpu.sync_copy(dstm_hbm.at[pl.ds(r, 16), :], db2)

                for j in range(16):
                    pltpu.async_copy(sh.at[sb2.at[j]],
                                     rows.at[pl.ds(j * 128, 128), :], sem)
                for j in range(16):
                    pltpu.make_async_copy(
                        sh.at[sb2.at[j]],
                        rows.at[pl.ds(j * 128, 128), :], sem).wait()
                for j in range(16):
                    pltpu.async_copy(sh.at[db2.at[j]],
                                     rows.at[pl.ds(j * 128, 128), :], sem,
                                     add=True)
                for j in range(16):
                    pltpu.make_async_copy(
                        sh.at[db2.at[j]],
                        rows.at[pl.ds(j * 128, 128), :], sem).wait()
                pltpu.sync_copy(rows,
                                out_hbm.at[pl.ds(c * EP + r * 128, 2048), :])

            def chunk_body(ci, _):
                chunk(ci)
                return 0
            lax.fori_loop(0, nch, chunk_body, 0)
            plsc.subcore_barrier()

    return k


def kernel(node_features, edge_features, node_positions, node_radii,
           edge_index, node_types, params):
    p = params

    nf = jnp.pad(node_features.astype(F32), ((0, NP - N_NODES), (0, 0)))
    types = jnp.pad(node_types.astype(I32), (0, NP - N_NODES)).reshape(NP, 1)
    pos = jnp.pad(node_positions.astype(F32), ((0, NP - N_NODES), (0, 0)))
    rad = jnp.pad(node_radii.astype(F32), (0, NP - N_NODES)).reshape(NP, 1)

    ei = edge_index.astype(I32)
    npad = EP - N_EDGES
    fill = N_NODES + (jnp.arange(npad, dtype=I32) % (NP - N_NODES))
    src = jnp.concatenate([ei[0], fill])
    dst = jnp.concatenate([ei[1], fill])
    srcm = src.reshape(EROWS, 128)
    dstm = dst.reshape(EROWS, 128)
    ef = jnp.pad(edge_features.astype(F32), ((0, npad), (0, 0)))

    mask = jnp.repeat(jnp.eye(HEADS, dtype=F32), HDIM, axis=0)
    zeros_tbl = jnp.zeros((NP, CW), F32)

    x = _tc_encode(nf, types, p['enc_node_W'], p['enc_node_b'].reshape(1, HID),
                   p['type_emb'])

    e_logits4 = _sc_edge_logits(4, "lrelu")
    e_logits1 = _sc_edge_logits(1, "sigmoid")
    seg_max = _sc_segment_max()
    exp_den = _sc_exp_den()
    wsc_gat = _sc_weighted_scatter(True)
    wsc_anat = _sc_weighted_scatter(False)
    pair_sum = _sc_pair_sum()

    topo = x
    topo_chunks = None
    for i in range(3):
        A8 = jnp.concatenate(
            [mask * p['gat%d_asrc' % i].reshape(-1, 1),
             mask * p['gat%d_adst' % i].reshape(-1, 1)], axis=1)
        h_tbl, s8 = _tc_layer_proj(topo, p['gat%d_W' % i], A8)
        sT = jnp.transpose(s8)
        e = e_logits4(sT[:4], sT[4:], srcm, dstm)
        m = seg_max(e, dstm)
        coef = exp_den(e, m, dstm)
        outf = wsc_gat(h_tbl.reshape(NC16 * NP, CW), coef,
                       srcm, dstm, zeros_tbl)
        res = _tc_elu_res(outf.reshape(NC16, NP, CW), topo,
                          with_chunks=(i == 2))
        topo = res[0]
        if i == 2:
            topo_chunks = res[1]

    ua, ub = _tc_geo(pos, rad, p['geo_W'].reshape(1, 4),
                     p['geo_b'].reshape(1, 1))
    gate = e_logits1(jnp.transpose(ua), jnp.transpose(ub), srcm, dstm)
    aggf = wsc_anat(topo_chunks.reshape(NC16 * NP, CW), gate,
                    srcm, dstm, zeros_tbl)
    nout, fchunks = _tc_fusion(aggf.reshape(NC16, NP, CW), topo, p)

    esumf = pair_sum(fchunks.reshape(NC16 * NP, CW), srcm, dstm)
    edge_op = _tc_edge_mlp(ef, esumf, p)

    return nout[:N_NODES], edge_op[:N_EDGES]


# speedup vs baseline: 18.3696x; 1.0311x over previous
import functools

import jax
import jax.numpy as jnp
from jax import lax
from jax.experimental import pallas as pl
from jax.experimental.pallas import tpu as pltpu
from jax.experimental.pallas import tpu_sc as plsc

F32 = jnp.float32
I32 = jnp.int32

N_NODES = 50000
NP = 51200
N_EDGES = 800000
EP = 819200
EROWS = EP // 128
D_NODE = 16
D_EDGE = 8
HID = 128
HEADS = 4
HDIM = 32
NC16 = 8
CW = 16

BLK = 1600
EBLK = 2048

_MESH = plsc.VectorSubcoreMesh(core_axis_name="c", subcore_axis_name="s")
NEG_BIG = -3.0e38

_GDN = lax.GatherDimensionNumbers(
    offset_dims=(), collapsed_slice_dims=(0,), start_index_map=(0,))


def _lane_gather(x, perm):
    return lax.gather(x, perm[:, None], _GDN, (1,),
                      mode=lax.GatherScatterMode.PROMISE_IN_BOUNDS)


def _enc_body(nf_ref, types_ref, w_ref, b_ref, temb_ref, out_ref):
    x = jnp.dot(nf_ref[...], w_ref[...], preferred_element_type=F32, precision=lax.Precision.HIGHEST)
    x = x + b_ref[...]
    t = types_ref[...]
    for k in range(3):
        x = x + jnp.where(t == k, temb_ref[k, :][None, :], 0.0)
    out_ref[...] = x


def _tc_encode(nf, types, W, b, temb):
    return pl.pallas_call(
        _enc_body,
        grid=(NP // BLK,),
        in_specs=[
            pl.BlockSpec((BLK, D_NODE), lambda i: (i, 0)),
            pl.BlockSpec((BLK, 1), lambda i: (i, 0)),
            pl.BlockSpec((D_NODE, HID), lambda i: (0, 0)),
            pl.BlockSpec((1, HID), lambda i: (0, 0)),
            pl.BlockSpec((3, HID), lambda i: (0, 0)),
        ],
        out_specs=pl.BlockSpec((BLK, HID), lambda i: (i, 0)),
        out_shape=jax.ShapeDtypeStruct((NP, HID), F32),
    )(nf, types, W, b, temb)


def _proj_body(x_ref, w_ref, a8_ref, h_ref, s_ref):
    h = jnp.dot(x_ref[...], w_ref[...], preferred_element_type=F32, precision=lax.Precision.HIGHEST)
    for c in range(NC16):
        h_ref[c] = h[:, c * CW:(c + 1) * CW]
    s_ref[...] = jnp.dot(h, a8_ref[...], preferred_element_type=F32, precision=lax.Precision.HIGHEST)


def _tc_layer_proj(x, W, A8):
    return pl.pallas_call(
        _proj_body,
        grid=(NP // BLK,),
        in_specs=[
            pl.BlockSpec((BLK, HID), lambda i: (i, 0)),
            pl.BlockSpec((HID, HID), lambda i: (0, 0)),
            pl.BlockSpec((HID, 8), lambda i: (0, 0)),
        ],
        out_specs=[
            pl.BlockSpec((NC16, BLK, CW), lambda i: (0, i, 0)),
            pl.BlockSpec((BLK, 8), lambda i: (i, 0)),
        ],
        out_shape=[
            jax.ShapeDtypeStruct((NC16, NP, CW), F32),
            jax.ShapeDtypeStruct((NP, 8), F32),
        ],
    )(x, W, A8)


def _elu_res_body(o_ref, t_ref, out_ref, chunks_ref):
    o = jnp.concatenate([o_ref[c] for c in range(NC16)], axis=-1)
    r = jnp.where(o > 0, o, jnp.exp(jnp.minimum(o, 0.0)) - 1.0) + t_ref[...]
    out_ref[...] = r
    if chunks_ref is not None:
        for c in range(NC16):
            chunks_ref[c] = r[:, c * CW:(c + 1) * CW]


def _tc_elu_res(out_tbl, topo, with_chunks):
    body = _elu_res_body if with_chunks else (
        lambda o_ref, t_ref, out_ref: _elu_res_body(o_ref, t_ref, out_ref, None))
    out_specs = [pl.BlockSpec((BLK, HID), lambda i: (i, 0))]
    out_shape = [jax.ShapeDtypeStruct((NP, HID), F32)]
    if with_chunks:
        out_specs.append(pl.BlockSpec((NC16, BLK, CW), lambda i: (0, i, 0)))
        out_shape.append(jax.ShapeDtypeStruct((NC16, NP, CW), F32))
    return pl.pallas_call(
        body,
        grid=(NP // BLK,),
        in_specs=[
            pl.BlockSpec((NC16, BLK, CW), lambda i: (0, i, 0)),
            pl.BlockSpec((BLK, HID), lambda i: (i, 0)),
        ],
        out_specs=out_specs,
        out_shape=out_shape,
    )(out_tbl, topo)


def _geo_body(pos_ref, rad_ref, gw_ref, gb_ref, ua_ref, ub_ref):
    u = jnp.sum(pos_ref[...] * gw_ref[0:1, 0:3], axis=1, keepdims=True)
    u = u + rad_ref[...] * gw_ref[0:1, 3:4]
    ua_ref[...] = u + gb_ref[...]
    ub_ref[...] = -u


def _tc_geo(pos, rad, gw_row, gb):
    return pl.pallas_call(
        _geo_body,
        grid=(NP // BLK,),
        in_specs=[
            pl.BlockSpec((BLK, 3), lambda i: (i, 0)),
            pl.BlockSpec((BLK, 1), lambda i: (i, 0)),
            pl.BlockSpec((1, 4), lambda i: (0, 0)),
            pl.BlockSpec((1, 1), lambda i: (0, 0)),
        ],
        out_specs=[
            pl.BlockSpec((BLK, 1), lambda i: (i, 0)),
            pl.BlockSpec((BLK, 1), lambda i: (i, 0)),
        ],
        out_shape=[
            jax.ShapeDtypeStruct((NP, 1), F32),
            jax.ShapeDtypeStruct((NP, 1), F32),
        ],
    )(pos, rad, gw_row, gb)


def _fusion_body(agg_ref, topo_ref, aw_ref, ab_ref, fw1_ref, fw2_ref, fb_ref,
                 lg_ref, lb_ref, nw1_ref, nb1_ref, nw2_ref, nb2_ref,
                 cw1_ref, cb1_ref, cw2_ref, cb2_ref, qw1_ref, qb1_ref,
                 qw2_ref, qb2_ref, nout_ref, fchunks_ref):
    agg = jnp.concatenate([agg_ref[c] for c in range(NC16)], axis=-1)
    topo = topo_ref[...]
    anat = jax.nn.relu(jnp.dot(agg, aw_ref[...], preferred_element_type=F32, precision=lax.Precision.HIGHEST)
                       + ab_ref[...]) + topo
    pre = jax.nn.relu(
        jnp.dot(topo, fw1_ref[...], preferred_element_type=F32, precision=lax.Precision.HIGHEST)
        + jnp.dot(anat, fw2_ref[...], preferred_element_type=F32, precision=lax.Precision.HIGHEST)
        + fb_ref[...])
    mu = jnp.mean(pre, axis=-1, keepdims=True)
    var = jnp.mean((pre - mu) ** 2, axis=-1, keepdims=True)
    fused = (pre - mu) / jnp.sqrt(var + 1e-5) * lg_ref[...] + lb_ref[...]
    nop = jnp.dot(jax.nn.relu(
        jnp.dot(fused, nw1_ref[...], preferred_element_type=F32, precision=lax.Precision.HIGHEST) + nb1_ref[...]),
        nw2_ref[...], preferred_element_type=F32, precision=lax.Precision.HIGHEST) + nb2_ref[...]
    ncr = jnp.dot(jax.nn.relu(
        jnp.dot(fused, cw1_ref[...], preferred_element_type=F32, precision=lax.Precision.HIGHEST) + cb1_ref[...]),
        cw2_ref[...], preferred_element_type=F32, precision=lax.Precision.HIGHEST) + cb2_ref[...]
    q = jax.nn.sigmoid(jnp.dot(jax.nn.relu(
        jnp.dot(fused, qw1_ref[...], preferred_element_type=F32, precision=lax.Precision.HIGHEST) + qb1_ref[...]),
        qw2_ref[...], preferred_element_type=F32, precision=lax.Precision.HIGHEST) + qb2_ref[...])
    nout_ref[...] = jnp.concatenate([nop, ncr, q], axis=-1)
    for c in range(NC16):
        fchunks_ref[c] = fused[:, c * CW:(c + 1) * CW]


def _tc_fusion(agg_tbl, topo, p):
    cst = lambda shp: pl.BlockSpec(shp, lambda i: tuple(0 for _ in shp))
    return pl.pallas_call(
        _fusion_body,
        grid=(NP // BLK,),
        in_specs=[
            pl.BlockSpec((NC16, BLK, CW), lambda i: (0, i, 0)),
            pl.BlockSpec((BLK, HID), lambda i: (i, 0)),
            cst((HID, HID)), cst((1, HID)),
            cst((HID, HID)), cst((HID, HID)), cst((1, HID)),
            cst((1, HID)), cst((1, HID)),
            cst((HID, 64)), cst((1, 64)), cst((64, 3)), cst((1, 3)),
            cst((HID, 64)), cst((1, 64)), cst((64, 7)), cst((1, 7)),
            cst((HID, 32)), cst((1, 32)), cst((32, 1)), cst((1, 1)),
        ],
        out_specs=[
            pl.BlockSpec((BLK, 11), lambda i: (i, 0)),
            pl.BlockSpec((NC16, BLK, CW), lambda i: (0, i, 0)),
        ],
        out_shape=[
            jax.ShapeDtypeStruct((NP, 11), F32),
            jax.ShapeDtypeStruct((NC16, NP, CW), F32),
        ],
    )(agg_tbl, topo,
      p['anat_W'], p['anat_b'].reshape(1, HID),
      p['fuse_W'][:HID], p['fuse_W'][HID:], p['fuse_b'].reshape(1, HID),
      p['ln_g'].reshape(1, HID), p['ln_b'].reshape(1, HID),
      p['nop_W1'], p['nop_b1'].reshape(1, 64), p['nop_W2'], p['nop_b2'].reshape(1, 3),
      p['ncr_W1'], p['ncr_b1'].reshape(1, 64), p['ncr_W2'], p['ncr_b2'].reshape(1, 7),
      p['q_W1'], p['q_b1'].reshape(1, 32), p['q_W2'], p['q_b2'].reshape(1, 1))


def _edge_mlp_body(ef_ref, e0_ref, e1_ref, e2_ref, e3_ref, e4_ref, e5_ref,
                   e6_ref, e7_ref, we_ref, be_ref,
                   w1_ref, b1_ref, w2_ref, b2_ref, out_ref):
    es = [e0_ref, e1_ref, e2_ref, e3_ref, e4_ref, e5_ref, e6_ref, e7_ref]
    ef = ef_ref[...]
    acc = jnp.zeros((ef.shape[0], 32), F32)
    for c in range(NC16):
        sl = slice(c * CW, (c + 1) * CW)
        eh_c = jax.nn.relu(
            jnp.dot(ef, we_ref[...][:, sl], preferred_element_type=F32, precision=lax.Precision.HIGHEST)
            + be_ref[...][:, sl])
        acc = acc + jnp.dot(eh_c + es[c][...], w1_ref[...][sl, :],
                            preferred_element_type=F32, precision=lax.Precision.HIGHEST)
    h1 = jax.nn.relu(acc + b1_ref[...])
    out_ref[...] = jnp.dot(h1, w2_ref[...], preferred_element_type=F32, precision=lax.Precision.HIGHEST) + b2_ref[...]


def _tc_edge_mlp(ef, esumf, p):
    cst = lambda shp: pl.BlockSpec(shp, lambda i: tuple(0 for _ in shp))
    nb = EP // EBLK
    espec = lambda c: pl.BlockSpec((EBLK, CW), lambda i, c=c: (c * nb + i, 0))
    return pl.pallas_call(
        _edge_mlp_body,
        grid=(nb,),
        in_specs=[
            pl.BlockSpec((EBLK, D_EDGE), lambda i: (i, 0)),
            espec(0), espec(1), espec(2), espec(3),
            espec(4), espec(5), espec(6), espec(7),
            cst((D_EDGE, HID)), cst((1, HID)),
            cst((HID, 32)), cst((1, 32)), cst((32, 3)), cst((1, 3)),
        ],
        out_specs=pl.BlockSpec((EBLK, 3), lambda i: (i, 0)),
        out_shape=jax.ShapeDtypeStruct((EP, 3), F32),
    )(ef, esumf, esumf, esumf, esumf, esumf, esumf, esumf, esumf,
      p['enc_edge_W'], p['enc_edge_b'].reshape(1, HID),
      p['eop_W1'], p['eop_b1'].reshape(1, 32), p['eop_W2'], p['eop_b2'].reshape(1, 3))


def _sc_edge_logits(n_heads, act):
    tph = 32 // n_heads
    rpt = EROWS // tph
    chr_ = 40
    nch = rpt // chr_

    @functools.partial(
        pl.kernel,
        compiler_params=pltpu.CompilerParams(
            needs_layout_passes=False, use_tc_tiling_on_sc=False),
        out_type=jax.ShapeDtypeStruct((n_heads, EROWS, 128), F32),
        mesh=_MESH,
        scratch_types=[
            pltpu.VMEM((NP,), F32), pltpu.VMEM((NP,), F32),
            pltpu.VMEM((40, 128), I32), pltpu.VMEM((40, 128), I32),
            pltpu.VMEM((40, 128), F32),
        ],
    )
    def k(sa_hbm, sb_hbm, srcm_hbm, dstm_hbm, e_hbm, cola, colb, sb2, db2, eb2):
        core = lax.axis_index("c")
        sub = lax.axis_index("s")
        if n_heads == 4:
            head = core * 2 + sub // 8
            slot = core * 0 + sub % 8
        else:
            head = core * 0 + sub * 0
            slot = core * 16 + sub
        pltpu.sync_copy(sa_hbm.at[head], cola)
        pltpu.sync_copy(sb_hbm.at[head], colb)
        row0 = slot * rpt

        def chunk(ci, _):
            r = row0 + ci * chr_
            pltpu.sync_copy(srcm_hbm.at[pl.ds(r, chr_), :], sb2)
            pltpu.sync_copy(dstm_hbm.at[pl.ds(r, chr_), :], db2)

            def row(j, _):
                for l in range(8):
                    isrc = sb2[j, pl.ds(l * 16, 16)]
                    idst = db2[j, pl.ds(l * 16, 16)]
                    a = plsc.load_gather(cola, [isrc])
                    b = plsc.load_gather(colb, [idst])
                    v = a + b
                    if act == "lrelu":
                        r_ = jnp.where(v >= 0, v, 0.2 * v)
                    else:
                        r_ = 1.0 / (1.0 + jnp.exp(-v))
                    eb2[j, pl.ds(l * 16, 16)] = r_
                return 0

            lax.fori_loop(0, chr_, row, 0)
            pltpu.sync_copy(eb2, e_hbm.at[head, pl.ds(r, chr_), :])
            return 0

        lax.fori_loop(0, nch, chunk, 0)

    return k


def _sc_segment_max():
    tph = 8
    rpt = EROWS // tph
    chr_ = 40
    nch = rpt // chr_

    @functools.partial(
        pl.kernel,
        compiler_params=pltpu.CompilerParams(
            needs_layout_passes=False, use_tc_tiling_on_sc=False),
        out_type=jax.ShapeDtypeStruct((HEADS, NP), F32),
        mesh=_MESH,
        scratch_types=[
            pltpu.VMEM((NP,), F32),
            pltpu.VMEM((40, 128), I32), pltpu.VMEM((40, 128), F32),
            pltpu.VMEM((3200,), F32), pltpu.VMEM((3200,), F32),
            pltpu.VMEM_SHARED((16, NP), F32),
        ],
    )
    def k(e_hbm, dstm_hbm, m_hbm, macc, db2, eb2, cbuf, tbuf, msh):
        core = lax.axis_index("c")
        sub = lax.axis_index("s")
        head = core * 2 + sub // 8
        slot = sub % 8
        neg = jnp.full((16,), NEG_BIG, F32)

        def ini(i, _):
            macc[pl.ds(i * 16, 16)] = neg
            return 0
        lax.fori_loop(0, NP // 16, ini, 0)

        iota = lax.iota(I32, 16)
        row0 = slot * rpt

        def chunk(ci, _):
            r = row0 + ci * chr_
            pltpu.sync_copy(dstm_hbm.at[pl.ds(r, chr_), :], db2)
            pltpu.sync_copy(e_hbm.at[head, pl.ds(r, chr_), :], eb2)

            def row(j, _):
                for l in range(8):
                    idx = db2[j, pl.ds(l * 16, 16)]
                    ev = eb2[j, pl.ds(l * 16, 16)]
                    acc = ev
                    for kk in range(1, 16):
                        perm = (iota + kk) & 15
                        oi = _lane_gather(idx, perm)
                        ov = _lane_gather(ev, perm)
                        acc = jnp.where(oi == idx, jnp.maximum(acc, ov), acc)
                    old = plsc.load_gather(macc, [idx])
                    plsc.store_scatter(macc, [idx], jnp.maximum(old, acc))
                return 0

            lax.fori_loop(0, chr_, row, 0)
            return 0

        lax.fori_loop(0, nch, chunk, 0)
        pltpu.sync_copy(macc, msh.at[sub])
        plsc.subcore_barrier()
        for g in range(2):
            pltpu.sync_copy(msh.at[g * 8, pl.ds(sub * 3200, 3200)], cbuf)
            for rcp in range(1, 8):
                pltpu.sync_copy(msh.at[g * 8 + rcp, pl.ds(sub * 3200, 3200)], tbuf)

                def mx(i, _):
                    cbuf[pl.ds(i * 16, 16)] = jnp.maximum(
                        cbuf[pl.ds(i * 16, 16)], tbuf[pl.ds(i * 16, 16)])
                    return 0
                lax.fori_loop(0, 200, mx, 0)
            pltpu.sync_copy(cbuf, m_hbm.at[core * 2 + g, pl.ds(sub * 3200, 3200)])
        plsc.subcore_barrier()

    return k


def _sc_exp_den():
    tph = 8
    rpt = EROWS // tph
    chr_ = 40
    nch = rpt // chr_

    @functools.partial(
        pl.kernel,
        compiler_params=pltpu.CompilerParams(
            needs_layout_passes=False, use_tc_tiling_on_sc=False),
        out_type=jax.ShapeDtypeStruct((HEADS, EROWS, 128), F32),
        mesh=_MESH,
        scratch_types=[
            pltpu.VMEM((NP,), F32),
            pltpu.VMEM((40, 128), I32), pltpu.VMEM((40, 128), I32),
            pltpu.VMEM((40, 128), F32), pltpu.VMEM((40, 128), F32),
            pltpu.VMEM((3200,), F32),
            pltpu.VMEM_SHARED((2 * NP,), F32),
            pltpu.SemaphoreType.DMA,
        ],
    )
    def k(e_hbm, m_hbm, dstm_hbm, ex_hbm,
          mcol, db2, ab2, eb2, xb2, zbuf, densh, dsem):
        core = lax.axis_index("c")
        sub = lax.axis_index("s")
        head = core * 2 + sub // 8
        lhead = sub // 8
        slot = sub % 8
        pltpu.sync_copy(m_hbm.at[head], mcol)

        def z(i, _):
            zbuf[pl.ds(i * 16, 16)] = jnp.zeros((16,), F32)
            return 0
        lax.fori_loop(0, 200, z, 0)
        pltpu.sync_copy(zbuf, densh.at[pl.ds(lhead * NP + slot * 6400, 3200)])
        pltpu.sync_copy(zbuf, densh.at[pl.ds(lhead * NP + slot * 6400 + 3200, 3200)])
        plsc.subcore_barrier()

        row0 = slot * rpt
        base = lhead * NP

        def chunk(ci, _):
            r = row0 + ci * chr_
            pltpu.sync_copy(dstm_hbm.at[pl.ds(r, chr_), :], db2)
            pltpu.sync_copy(e_hbm.at[head, pl.ds(r, chr_), :], eb2)

            def row(j, _):
                for l in range(8):
                    idx = db2[j, pl.ds(l * 16, 16)]
                    ev = eb2[j, pl.ds(l * 16, 16)]
                    mv = plsc.load_gather(mcol, [idx])
                    xb2[j, pl.ds(l * 16, 16)] = jnp.exp(ev - mv)
                    ab2[j, pl.ds(l * 16, 16)] = idx + base
                return 0

            lax.fori_loop(0, chr_, row, 0)
            pltpu.sync_copy(xb2, ex_hbm.at[head, pl.ds(r, chr_), :])

            for j in range(40):
                pltpu.async_copy(xb2.at[j], densh.at[ab2.at[j]], dsem, add=True)
            for j in range(40):
                pltpu.make_async_copy(xb2.at[j], densh.at[ab2.at[j]],
                                      dsem).wait()
            return 0

        lax.fori_loop(0, nch, chunk, 0)
        plsc.subcore_barrier()
        pltpu.sync_copy(densh.at[pl.ds(lhead * NP, NP)], mcol)

        def chunk2(ci, _):
            r = row0 + ci * chr_
            pltpu.sync_copy(dstm_hbm.at[pl.ds(r, chr_), :], db2)
            pltpu.sync_copy(ex_hbm.at[head, pl.ds(r, chr_), :], xb2)

            def row2(j, _):
                for l in range(8):
                    idx = db2[j, pl.ds(l * 16, 16)]
                    xv = xb2[j, pl.ds(l * 16, 16)]
                    dv = plsc.load_gather(mcol, [idx])
                    xb2[j, pl.ds(l * 16, 16)] = xv / (dv + 1e-9)
                return 0

            lax.fori_loop(0, chr_, row2, 0)
            pltpu.sync_copy(xb2, ex_hbm.at[head, pl.ds(r, chr_), :])
            return 0

        lax.fori_loop(0, nch, chunk2, 0)
        plsc.subcore_barrier()

    return k


def _sc_weighted_scatter(head_w):
    rpt = EROWS // 16
    nch = rpt // 16

    scratch = [
        pltpu.VMEM((16, 128), I32), pltpu.VMEM((16, 128), I32),
        pltpu.VMEM((16, 128), I32), pltpu.VMEM((16, 128), F32),
        pltpu.VMEM((2048, CW), F32),
        pltpu.VMEM_SHARED((NP, CW), F32),
        pltpu.SemaphoreType.DMA,
    ]

    @functools.partial(
        pl.kernel,
        compiler_params=pltpu.CompilerParams(
            needs_layout_passes=False, use_tc_tiling_on_sc=False),
        out_type=jax.ShapeDtypeStruct((NC16 * NP, CW), F32),
        mesh=_MESH,
        scratch_types=scratch,
    )
    def k(tblf_hbm, w_hbm, srcm_hbm, dstm_hbm, zer_hbm, out_hbm, *scr):
        sb2, db2, ab2, wb2, rows, acc, sem = scr
        core = lax.axis_index("c")
        sub = lax.axis_index("s")
        row0 = sub * rpt

        for li in range(4):
            c = core * 4 + li
            wrow = (c >> 1) if head_w else c * 0
            pltpu.sync_copy(zer_hbm.at[pl.ds(sub * 3200, 3200), :],
                            acc.at[pl.ds(sub * 3200, 3200), :])
            plsc.subcore_barrier()

            def chunk(ci, c=c, wrow=wrow):
                r = row0 + ci * 16
                pltpu.sync_copy(srcm_hbm.at[pl.ds(r, 16), :], sb2)
                pltpu.sync_copy(dstm_hbm.at[pl.ds(r, 16), :], db2)
                pltpu.sync_copy(w_hbm.at[wrow, pl.ds(r, 16), :], wb2)

                def adj(j, _):
                    for l in range(8):
                        ab2[j, pl.ds(l * 16, 16)] = (
                            sb2[j, pl.ds(l * 16, 16)] + c * NP)
                    return 0
                lax.fori_loop(0, 16, adj, 0)

                for j in range(16):
                    pltpu.async_copy(tblf_hbm.at[ab2.at[j]],
                                     rows.at[pl.ds(j * 128, 128), :], sem)
                for j in range(16):
                    pltpu.make_async_copy(
                        tblf_hbm.at[ab2.at[j]],
                        rows.at[pl.ds(j * 128, 128), :], sem).wait()

                def scale(j, _):
                    for l in range(8):
                        coef = wb2[j, pl.ds(l * 16, 16)]
                        for jj in range(16):
                            e_in_row = l * 16 + jj
                            cj = coef[jj]
                            r0 = rows[j * 128 + e_in_row, pl.ds(0, 16)]
                            rows[j * 128 + e_in_row, pl.ds(0, 16)] = r0 * cj
                    return 0
                lax.fori_loop(0, 16, scale, 0)

                for j in range(16):
                    pltpu.async_copy(rows.at[pl.ds(j * 128, 128), :],
                                     acc.at[db2.at[j]], sem, add=True)
                for j in range(16):
                    pltpu.make_async_copy(
                        rows.at[pl.ds(j * 128, 128), :],
                        acc.at[db2.at[j]], sem).wait()

            def chunk_body(ci, _):
                chunk(ci)
                return 0
            lax.fori_loop(0, nch, chunk_body, 0)
            plsc.subcore_barrier()
            pltpu.sync_copy(
                acc.at[pl.ds(sub * 3200, 3200), :],
                out_hbm.at[pl.ds(c * NP + sub * 3200, 3200), :])
            plsc.subcore_barrier()

    return k


def _sc_pair_sum():
    rpt = EROWS // 16
    nch = rpt // 16

    @functools.partial(
        pl.kernel,
        compiler_params=pltpu.CompilerParams(
            needs_layout_passes=False, use_tc_tiling_on_sc=False),
        out_type=jax.ShapeDtypeStruct((NC16 * EP, CW), F32),
        mesh=_MESH,
        scratch_types=[
            pltpu.VMEM((16, 128), I32), pltpu.VMEM((16, 128), I32),
            pltpu.VMEM((2048, CW), F32),
            pltpu.VMEM_SHARED((NP, CW), F32),
            pltpu.SemaphoreType.DMA,
        ],
    )
    def k(tblf_hbm, srcm_hbm, dstm_hbm, out_hbm, sb2, db2, rows, sh, sem):
        core = lax.axis_index("c")
        sub = lax.axis_index("s")
        row0 = sub * rpt

        for li in range(4):
            c = core * 4 + li
            pltpu.sync_copy(
                tblf_hbm.at[pl.ds(c * NP + sub * 3200, 3200), :],
                sh.at[pl.ds(sub * 3200, 3200), :])
            plsc.subcore_barrier()

            def chunk(ci, c=c):
                r = row0 + ci * 16
                pltpu.sync_copy(srcm_hbm.at[pl.ds(r, 16), :], sb2)
                pltpu.sync_copy(dstm_hbm.at[pl.ds(r, 16), :], db2)

                for j in range(16):
                    pltpu.async_copy(sh.at[sb2.at[j]],
                                     rows.at[pl.ds(j * 128, 128), :], sem)
                for j in range(16):
                    pltpu.make_async_copy(
                        sh.at[sb2.at[j]],
                        rows.at[pl.ds(j * 128, 128), :], sem).wait()
                for j in range(16):
                    pltpu.async_copy(sh.at[db2.at[j]],
                                     rows.at[pl.ds(j * 128, 128), :], sem,
                                     add=True)
                for j in range(16):
                    pltpu.make_async_copy(
                        sh.at[db2.at[j]],
                        rows.at[pl.ds(j * 128, 128), :], sem).wait()
                pltpu.sync_copy(rows,
                                out_hbm.at[pl.ds(c * EP + r * 128, 2048), :])

            def chunk_body(ci, _):
                chunk(ci)
                return 0
            lax.fori_loop(0, nch, chunk_body, 0)
            plsc.subcore_barrier()

    return k


def kernel(node_features, edge_features, node_positions, node_radii,
           edge_index, node_types, params):
    p = params

    nf = jnp.pad(node_features.astype(F32), ((0, NP - N_NODES), (0, 0)))
    types = jnp.pad(node_types.astype(I32), (0, NP - N_NODES)).reshape(NP, 1)
    pos = jnp.pad(node_positions.astype(F32), ((0, NP - N_NODES), (0, 0)))
    rad = jnp.pad(node_radii.astype(F32), (0, NP - N_NODES)).reshape(NP, 1)

    ei = edge_index.astype(I32)
    npad = EP - N_EDGES
    fill = N_NODES + (jnp.arange(npad, dtype=I32) % (NP - N_NODES))
    src = jnp.concatenate([ei[0], fill])
    dst = jnp.concatenate([ei[1], fill])
    srcm = src.reshape(EROWS, 128)
    dstm = dst.reshape(EROWS, 128)
    ef = jnp.pad(edge_features.astype(F32), ((0, npad), (0, 0)))

    mask = jnp.repeat(jnp.eye(HEADS, dtype=F32), HDIM, axis=0)
    zeros_tbl = jnp.zeros((NP, CW), F32)

    x = _tc_encode(nf, types, p['enc_node_W'], p['enc_node_b'].reshape(1, HID),
                   p['type_emb'])

    e_logits4 = _sc_edge_logits(4, "lrelu")
    e_logits1 = _sc_edge_logits(1, "sigmoid")
    seg_max = _sc_segment_max()
    exp_den = _sc_exp_den()
    wsc_gat = _sc_weighted_scatter(True)
    wsc_anat = _sc_weighted_scatter(False)
    pair_sum = _sc_pair_sum()

    topo = x
    topo_chunks = None
    for i in range(3):
        A8 = jnp.concatenate(
            [mask * p['gat%d_asrc' % i].reshape(-1, 1),
             mask * p['gat%d_adst' % i].reshape(-1, 1)], axis=1)
        h_tbl, s8 = _tc_layer_proj(topo, p['gat%d_W' % i], A8)
        sT = jnp.transpose(s8)
        e = e_logits4(sT[:4], sT[4:], srcm, dstm)
        m = seg_max(e, dstm)
        coef = exp_den(e, m, dstm)
        outf = wsc_gat(h_tbl.reshape(NC16 * NP, CW), coef,
                       srcm, dstm, zeros_tbl)
        res = _tc_elu_res(outf.reshape(NC16, NP, CW), topo,
                          with_chunks=(i == 2))
        topo = res[0]
        if i == 2:
            topo_chunks = res[1]

    ua, ub = _tc_geo(pos, rad, p['geo_W'].reshape(1, 4),
                     p['geo_b'].reshape(1, 1))
    gate = e_logits1(jnp.transpose(ua), jnp.transpose(ub), srcm, dstm)
    aggf = wsc_anat(topo_chunks.reshape(NC16 * NP, CW), gate,
                    srcm, dstm, zeros_tbl)
    nout, fchunks = _tc_fusion(aggf.reshape(NC16, NP, CW), topo, p)

    esumf = pair_sum(fchunks.reshape(NC16 * NP, CW), srcm, dstm)
    edge_op = _tc_edge_mlp(ef, esumf, p)

    return nout[:N_NODES], edge_op[:N_EDGES]
